# baseline (speedup 1.0000x reference)
import jax
import jax.numpy as jnp
from jax import lax
from jax.experimental import pallas as pl
from jax.experimental.pallas import tpu as pltpu

_BS = 32
_DX = 512
_DY = 512
_NPT = _DX * _DY
_NCH = _NPT // 128
_NR = 2 * _NCH
_GB = 8
_RH = 4
_RB = _NR // _RH
_XR = _DX // _RH


def _body(xl_ref, yl_ref, hist_ref, a_ref, w_ref):
    rh = pl.program_id(1)
    sub = lax.broadcasted_iota(jnp.int32, (8, 128), 0)
    lane = lax.broadcasted_iota(jnp.int32, (8, 128), 1)
    ypat = (((sub >> 1) & 3) << 7) + lane
    ypat_f = ypat.astype(jnp.float32)
    m8 = (sub & 1) == 0
    ng = _RB // 8
    mfull = jnp.broadcast_to(m8[None], (ng, 8, 128)).reshape(_RB, 128)
    q = rh * _XR + (lax.broadcasted_iota(jnp.int32, (_RB, 1), 0) >> 3)
    qf = q.astype(jnp.float32)

    for i in range(_GB):
        b = pl.program_id(0) * _GB + i
        x0 = xl_ref[b, 0]
        y0 = yl_ref[b, 0]
        wx = (xl_ref[b, 1] - x0) * (1.0 / _DX)
        wy = (yl_ref[b, 1] - y0) * (1.0 / _DY)
        bx = x0 + wx * 0.5
        by = y0 + wy * 0.5
        y8 = ypat_f * wy + by
        yfull = jnp.broadcast_to(y8[None], (ng, 8, 128)).reshape(_RB, 128)
        xcol = qf * wx + bx
        a_ref[i] = jnp.where(mfull, jnp.broadcast_to(xcol, (_RB, 128)), yfull)

    w_ref[...] = hist_ref[...].reshape(_GB, _XR * _DY)


def kernel(hist, x_lims, y_lims):
    a, w = pl.pallas_call(
        _body,
        grid=(_BS // _GB, _RH),
        in_specs=[
            pl.BlockSpec(memory_space=pltpu.SMEM),
            pl.BlockSpec(memory_space=pltpu.SMEM),
            pl.BlockSpec((_GB, _XR, _DY), lambda g, r: (g, r, 0)),
        ],
        out_specs=[
            pl.BlockSpec((_GB, _RB, 128), lambda g, r: (g, r, 0)),
            pl.BlockSpec((_GB, _XR * _DY), lambda g, r: (g, r)),
        ],
        out_shape=[
            jax.ShapeDtypeStruct((_BS, _NR, 128), jnp.float32),
            jax.ShapeDtypeStruct((_BS, _NPT), jnp.float32),
        ],
    )(x_lims, y_lims, hist)
    pc = a.reshape(_BS, _NCH, 2, 128).transpose(0, 1, 3, 2).reshape(_BS, _NPT, 2)
    return pc, w


# speedup vs baseline: 1.0437x; 1.0437x over previous
import jax
import jax.numpy as jnp
from jax import lax
from jax.experimental import pallas as pl
from jax.experimental.pallas import tpu as pltpu

_BS = 32
_DX = 512
_DY = 512
_NPT = _DX * _DY
_NCH = _NPT // 128
_NR = 2 * _NCH
_GB = 8
_RH = 2
_RB = _NR // _RH
_XR = _DX // _RH


def _body(xl_ref, yl_ref, hist_ref, a_ref, w_ref):
    rh = pl.program_id(1)
    sub = lax.broadcasted_iota(jnp.int32, (8, 128), 0)
    lane = lax.broadcasted_iota(jnp.int32, (8, 128), 1)
    ypat = (((sub >> 1) & 3) << 7) + lane
    ypat_f = ypat.astype(jnp.float32)
    m8 = (sub & 1) == 0
    ng = _RB // 8
    mfull = jnp.broadcast_to(m8[None], (ng, 8, 128)).reshape(_RB, 128)
    q = rh * _XR + (lax.broadcasted_iota(jnp.int32, (_RB, 1), 0) >> 3)
    qf = q.astype(jnp.float32)

    for i in range(_GB):
        b = pl.program_id(0) * _GB + i
        x0 = xl_ref[b, 0]
        y0 = yl_ref[b, 0]
        wx = (xl_ref[b, 1] - x0) * (1.0 / _DX)
        wy = (yl_ref[b, 1] - y0) * (1.0 / _DY)
        bx = x0 + wx * 0.5
        by = y0 + wy * 0.5
        y8 = ypat_f * wy + by
        yfull = jnp.broadcast_to(y8[None], (ng, 8, 128)).reshape(_RB, 128)
        xcol = qf * wx + bx
        a_ref[i] = jnp.where(mfull, jnp.broadcast_to(xcol, (_RB, 128)), yfull)

    w_ref[...] = hist_ref[...].reshape(_GB, _XR * _DY)


def kernel(hist, x_lims, y_lims):
    a, w = pl.pallas_call(
        _body,
        grid=(_BS // _GB, _RH),
        in_specs=[
            pl.BlockSpec(memory_space=pltpu.SMEM),
            pl.BlockSpec(memory_space=pltpu.SMEM),
            pl.BlockSpec((_GB, _XR, _DY), lambda g, r: (g, r, 0)),
        ],
        out_specs=[
            pl.BlockSpec((_GB, _RB, 128), lambda g, r: (g, r, 0)),
            pl.BlockSpec((_GB, _XR * _DY), lambda g, r: (g, r)),
        ],
        out_shape=[
            jax.ShapeDtypeStruct((_BS, _NR, 128), jnp.float32),
            jax.ShapeDtypeStruct((_BS, _NPT), jnp.float32),
        ],
    )(x_lims, y_lims, hist)
    pc = a.reshape(_BS, _NCH, 2, 128).transpose(0, 1, 3, 2).reshape(_BS, _NPT, 2)
    return pc, w


# speedup vs baseline: 1.0474x; 1.0035x over previous
import jax
import jax.numpy as jnp
from jax import lax
from jax.experimental import pallas as pl
from jax.experimental.pallas import tpu as pltpu

_BS = 32
_DX = 512
_DY = 512
_NPT = _DX * _DY
_NCH = _NPT // 128
_NR = 2 * _NCH
_GB = 8
_RH = 2
_RB = _NR // _RH
_XR = _DX // _RH


def _body(xl_ref, yl_ref, hist_ref, a_ref, w_ref):
    rh = pl.program_id(0)
    sub = lax.broadcasted_iota(jnp.int32, (8, 128), 0)
    lane = lax.broadcasted_iota(jnp.int32, (8, 128), 1)
    ypat = (((sub >> 1) & 3) << 7) + lane
    ypat_f = ypat.astype(jnp.float32)
    m8 = (sub & 1) == 0
    ng = _RB // 8
    mfull = jnp.broadcast_to(m8[None], (ng, 8, 128)).reshape(_RB, 128)
    q = rh * _XR + (lax.broadcasted_iota(jnp.int32, (_RB, 1), 0) >> 3)
    qf = q.astype(jnp.float32)

    for i in range(_GB):
        b = pl.program_id(1) * _GB + i
        x0 = xl_ref[b, 0]
        y0 = yl_ref[b, 0]
        wx = (xl_ref[b, 1] - x0) * (1.0 / _DX)
        wy = (yl_ref[b, 1] - y0) * (1.0 / _DY)
        bx = x0 + wx * 0.5
        by = y0 + wy * 0.5
        y8 = ypat_f * wy + by
        yfull = jnp.broadcast_to(y8[None], (ng, 8, 128)).reshape(_RB, 128)
        xcol = qf * wx + bx
        a_ref[i] = jnp.where(mfull, jnp.broadcast_to(xcol, (_RB, 128)), yfull)

    w_ref[...] = hist_ref[...].reshape(_GB, _XR * _DY)


def kernel(hist, x_lims, y_lims):
    a, w = pl.pallas_call(
        _body,
        grid=(_RH, _BS // _GB),
        in_specs=[
            pl.BlockSpec(memory_space=pltpu.SMEM),
            pl.BlockSpec(memory_space=pltpu.SMEM),
            pl.BlockSpec((_GB, _XR, _DY), lambda r, g: (g, r, 0)),
        ],
        out_specs=[
            pl.BlockSpec((_GB, _RB, 128), lambda r, g: (g, r, 0)),
            pl.BlockSpec((_GB, _XR * _DY), lambda r, g: (g, r)),
        ],
        out_shape=[
            jax.ShapeDtypeStruct((_BS, _NR, 128), jnp.float32),
            jax.ShapeDtypeStruct((_BS, _NPT), jnp.float32),
        ],
    )(x_lims, y_lims, hist)
    pc = a.reshape(_BS, _NCH, 2, 128).transpose(0, 1, 3, 2).reshape(_BS, _NPT, 2)
    return pc, w
